# baseline (speedup 1.0000x reference)
import jax
import jax.numpy as jnp
from jax import lax
from jax.experimental import pallas as pl
from jax.experimental.pallas import tpu as pltpu
from jax.experimental.pallas import tpu_sc as plsc

_N_USERS = 100000
_B = 4096
_D = 64
_KU = 20
_KW = 50
_NWORK = 32
_RPW = _B // _NWORK
_JC = 5
_NCU = _KU // _JC
_NCW = _KW // _JC
_ROWS_U = _N_USERS + 1
_CBLK = 8192
_NPAD = 106496
_CGRID = _NPAD // _CBLK

_SC_MESH = dict(
    mesh=plsc.VectorSubcoreMesh(core_axis_name="c", subcore_axis_name="s",
                                num_cores=2, num_subcores=16),
    compiler_params=pltpu.CompilerParams(needs_layout_passes=False,
                                         use_tc_tiling_on_sc=False),
)


def _relayout(x):
    xt = x.T
    xt3 = xt.reshape(_CBLK // 2, 2, _D)
    return jnp.concatenate([xt3[:, 0, :], xt3[:, 1, :]], axis=1)


def _convu_body(xt_ref, flat_ref, csum_ref):
    i = pl.program_id(0)

    @pl.when(i == 0)
    def _init():
        csum_ref[...] = jnp.zeros_like(csum_ref)

    x = xt_ref[...]

    @pl.when(i < _CGRID - 1)
    def _full():
        csum_ref[...] += jnp.sum(x, axis=1)

    @pl.when(i == _CGRID - 1)
    def _tail():
        cols = i * _CBLK + lax.broadcasted_iota(jnp.int32, x.shape, 1)
        csum_ref[...] += jnp.sum(jnp.where(cols < _ROWS_U, x, 0.0), axis=1)

    flat_ref[...] = _relayout(x)


def _convw_body(xt_ref, flat_ref):
    flat_ref[...] = _relayout(xt_ref[...])


def _convu(XT):
    return pl.pallas_call(
        _convu_body,
        grid=(_CGRID,),
        in_specs=[pl.BlockSpec((_D, _CBLK), lambda i: (0, i))],
        out_specs=(pl.BlockSpec((_CBLK // 2, 2 * _D), lambda i: (i, 0)),
                   pl.BlockSpec((_D,), lambda i: (0,))),
        out_shape=(jax.ShapeDtypeStruct((_NPAD // 2, 2 * _D), jnp.float32),
                   jax.ShapeDtypeStruct((_D,), jnp.float32)),
    )(XT)


def _convw(XT):
    return pl.pallas_call(
        _convw_body,
        grid=(_CGRID,),
        in_specs=[pl.BlockSpec((_D, _CBLK), lambda i: (0, i))],
        out_specs=pl.BlockSpec((_CBLK // 2, 2 * _D), lambda i: (i, 0)),
        out_shape=jax.ShapeDtypeStruct((_NPAD // 2, 2 * _D), jnp.float32),
    )(XT)


def _gather_accumulate(idx, tab, rows, acc, sem, nchunks):
    def _zero(r, c):
        z = jnp.zeros((16,), jnp.float32)
        for k in range(4):
            acc[pl.ds(r * _D + k * 16, 16)] = z
        return c
    lax.fori_loop(0, _RPW, _zero, 0)

    def _start(ci, buf):
        return [
            pltpu.async_copy(tab.at[idx.at[ci * _JC + j]], rows.at[buf, j],
                             sem)
            for j in range(_JC)
        ]

    inflight = _start(0, 0)
    for ci in range(nchunks):
        nxt = _start(ci + 1, (ci + 1) % 2) if ci + 1 < nchunks else None
        for h in inflight:
            h.wait()
        inflight = nxt
        buf = ci % 2

        def _acc(r, c):
            for k in range(4):
                sl = pl.ds(k * 16, 16)
                fl = pl.ds(r * _D + k * 16, 16)
                v = rows[buf, 0, r, sl]
                for j in range(1, _JC):
                    v = v + rows[buf, j, r, sl]
                acc[fl] = acc[fl] + v
            return c
        lax.fori_loop(0, _RPW, _acc, 0)


def _scw_body(wiT, W_hbm, accw_hbm, idxw, rows, accw, sem):
    cid = lax.axis_index("c")
    sid = lax.axis_index("s")
    wid = sid * 2 + cid
    base = wid * _RPW

    pltpu.sync_copy(wiT.at[:, pl.ds(base, _RPW)], idxw)
    _gather_accumulate(idxw, W_hbm, rows, accw, sem, _NCW)
    pltpu.sync_copy(accw, accw_hbm.at[pl.ds(base * _D, _RPW * _D)])


def _scu_body(uiT, l_hbm, n_hbm, U_hbm, usum_hbm, accw_hbm, s_hbm, sp_hbm,
              idxu, rows, accu, accw, lv, nv, usv, sv, spv, sem):
    cid = lax.axis_index("c")
    sid = lax.axis_index("s")
    wid = sid * 2 + cid
    base = wid * _RPW

    pltpu.sync_copy(uiT.at[:, pl.ds(base, _RPW)], idxu)
    pltpu.sync_copy(l_hbm.at[pl.ds(base, _RPW)], lv)
    pltpu.sync_copy(n_hbm.at[pl.ds(base, _RPW)], nv)
    pltpu.sync_copy(usum_hbm, usv)
    pltpu.sync_copy(accw_hbm.at[pl.ds(base * _D, _RPW * _D)], accw)
    _gather_accumulate(idxu, U_hbm, rows, accu, sem, _NCU)

    riota = lax.iota(jnp.int32, 16)
    zero16 = jnp.zeros((16,), jnp.float32)
    for g in range(_RPW // 16):
        ridx = riota + g * 16

        def _dbody(d, carry):
            du, dt = carry
            dvec = jnp.broadcast_to(d, (16,))
            fidx = ridx * _D + dvec
            gu = plsc.load_gather(accu, [fidx])
            gw = plsc.load_gather(accw, [fidx])
            us = plsc.load_gather(usv, [dvec])
            return (du + gu * gw, dt + us * gw)

        du, dt = lax.fori_loop(0, _D, _dbody, (zero16, zero16))
        sl = pl.ds(g * 16, 16)
        nf = nv[sl].astype(jnp.float32)
        lf = lv[sl].astype(jnp.float32)
        sv[sl] = du / (nf * lf)
        spv[sl] = (dt - du) / ((float(_N_USERS) - nf) * lf)

    pltpu.sync_copy(sv, s_hbm.at[pl.ds(base, _RPW)])
    pltpu.sync_copy(spv, sp_hbm.at[pl.ds(base, _RPW)])


_SCRATCH_W = [
    pltpu.VMEM((_KW, _RPW), jnp.int32),
    pltpu.VMEM((2, _JC, _RPW, _D), jnp.float32),
    pltpu.VMEM((_RPW * _D,), jnp.float32),
    pltpu.SemaphoreType.DMA,
]

_SCRATCH_U = [
    pltpu.VMEM((_KU, _RPW), jnp.int32),
    pltpu.VMEM((2, _JC, _RPW, _D), jnp.float32),
    pltpu.VMEM((_RPW * _D,), jnp.float32),
    pltpu.VMEM((_RPW * _D,), jnp.float32),
    pltpu.VMEM((_RPW,), jnp.int32),
    pltpu.VMEM((_RPW,), jnp.int32),
    pltpu.VMEM((_D,), jnp.float32),
    pltpu.VMEM((_RPW,), jnp.float32),
    pltpu.VMEM((_RPW,), jnp.float32),
    pltpu.SemaphoreType.DMA,
]


def kernel(ui, wi, l, n, U, W):
    w_lin = _convw(W.T)
    scw = pl.kernel(
        _scw_body,
        out_type=jax.ShapeDtypeStruct((_B * _D,), jnp.float32),
        scratch_types=_SCRATCH_W,
        **_SC_MESH,
    )
    accw = scw(wi.T, w_lin.reshape(_NPAD, _D))

    u_lin, usum = _convu(U.T)
    scu = pl.kernel(
        _scu_body,
        out_type=(jax.ShapeDtypeStruct((_B,), jnp.float32),
                  jax.ShapeDtypeStruct((_B,), jnp.float32)),
        scratch_types=_SCRATCH_U,
        **_SC_MESH,
    )
    s, sp = scu(ui.T, l, n, u_lin.reshape(_NPAD, _D), usum, accw)
    return (s, sp)


# speedup vs baseline: 1.0297x; 1.0297x over previous
import jax
import jax.numpy as jnp
from jax import lax
from jax.experimental import pallas as pl
from jax.experimental.pallas import tpu as pltpu
from jax.experimental.pallas import tpu_sc as plsc

_N_USERS = 100000
_B = 4096
_D = 64
_KU = 20
_KW = 50
_NWORK = 32
_RPW = _B // _NWORK
_JC = 5
_NCU = _KU // _JC
_NCW = _KW // _JC
_ROWS_U = _N_USERS + 1
_CBLK = 8192
_NPAD = 106496
_CGRID = _NPAD // _CBLK

_SC_MESH = dict(
    mesh=plsc.VectorSubcoreMesh(core_axis_name="c", subcore_axis_name="s",
                                num_cores=2, num_subcores=16),
    compiler_params=pltpu.CompilerParams(needs_layout_passes=False,
                                         use_tc_tiling_on_sc=False),
)


def _relayout(x):
    xt = x.T
    xt3 = xt.reshape(_CBLK // 2, 2, _D)
    return jnp.concatenate([xt3[:, 0, :], xt3[:, 1, :]], axis=1)


def _convu_body(xt_ref, flat_ref, csum_ref):
    i = pl.program_id(0)

    @pl.when(i == 0)
    def _init():
        csum_ref[...] = jnp.zeros_like(csum_ref)

    x = xt_ref[...]
    cols = i * _CBLK + lax.broadcasted_iota(jnp.int32, x.shape, 1)
    x = jnp.where(cols < _ROWS_U, x, 0.0)
    csum_ref[...] += jnp.sum(x, axis=1, keepdims=True)
    flat_ref[...] = _relayout(x)


def _convw_body(xt_ref, flat_ref):
    flat_ref[...] = _relayout(xt_ref[...])


def _convu(XT):
    return pl.pallas_call(
        _convu_body,
        grid=(_CGRID,),
        in_specs=[pl.BlockSpec((_D, _CBLK), lambda i: (0, i))],
        out_specs=(pl.BlockSpec((_CBLK // 2, 2 * _D), lambda i: (i, 0)),
                   pl.BlockSpec((_D, 1), lambda i: (0, 0))),
        out_shape=(jax.ShapeDtypeStruct((_NPAD // 2, 2 * _D), jnp.float32),
                   jax.ShapeDtypeStruct((_D, 1), jnp.float32)),
    )(XT)


def _convw(XT):
    return pl.pallas_call(
        _convw_body,
        grid=(_CGRID,),
        in_specs=[pl.BlockSpec((_D, _CBLK), lambda i: (0, i))],
        out_specs=pl.BlockSpec((_CBLK // 2, 2 * _D), lambda i: (i, 0)),
        out_shape=jax.ShapeDtypeStruct((_NPAD // 2, 2 * _D), jnp.float32),
    )(XT)


def _gather_accumulate(idx, tab, rows, acc, sem, nchunks, prelude=None):
    def _start(ci, buf):
        return [
            pltpu.async_copy(tab.at[idx.at[ci * _JC + j]], rows.at[buf, j],
                             sem)
            for j in range(_JC)
        ]

    inflight = _start(0, 0)
    if prelude is not None:
        prelude()

    def _zero(r, c):
        z = jnp.zeros((16,), jnp.float32)
        for k in range(4):
            acc[pl.ds(r * _D + k * 16, 16)] = z
        return c
    lax.fori_loop(0, _RPW, _zero, 0)
    for ci in range(nchunks):
        nxt = _start(ci + 1, (ci + 1) % 2) if ci + 1 < nchunks else None
        for h in inflight:
            h.wait()
        inflight = nxt
        buf = ci % 2

        def _acc(r, c):
            for k in range(4):
                sl = pl.ds(k * 16, 16)
                fl = pl.ds(r * _D + k * 16, 16)
                v = rows[buf, 0, r, sl]
                for j in range(1, _JC):
                    v = v + rows[buf, j, r, sl]
                acc[fl] = acc[fl] + v
            return c
        lax.fori_loop(0, _RPW, _acc, 0)


def _scw_body(wiT, W_hbm, accw_hbm, idxw, rows, accw, sem):
    cid = lax.axis_index("c")
    sid = lax.axis_index("s")
    wid = sid * 2 + cid
    base = wid * _RPW

    pltpu.sync_copy(wiT.at[:, pl.ds(base, _RPW)], idxw)
    _gather_accumulate(idxw, W_hbm, rows, accw, sem, _NCW)
    pltpu.sync_copy(accw, accw_hbm.at[pl.ds(base * _D, _RPW * _D)])


def _scu_body(uiT, l_hbm, n_hbm, U_hbm, usum_hbm, accw_hbm, s_hbm, sp_hbm,
              idxu, rows, accu, accw, lv, nv, usv, sv, spv, sem):
    cid = lax.axis_index("c")
    sid = lax.axis_index("s")
    wid = sid * 2 + cid
    base = wid * _RPW

    pltpu.sync_copy(uiT.at[:, pl.ds(base, _RPW)], idxu)

    def _prelude():
        pltpu.sync_copy(l_hbm.at[pl.ds(base, _RPW)], lv)
        pltpu.sync_copy(n_hbm.at[pl.ds(base, _RPW)], nv)
        pltpu.sync_copy(usum_hbm, usv)
        pltpu.sync_copy(accw_hbm.at[pl.ds(base * _D, _RPW * _D)], accw)
    _gather_accumulate(idxu, U_hbm, rows, accu, sem, _NCU, prelude=_prelude)

    riota = lax.iota(jnp.int32, 16)
    zero16 = jnp.zeros((16,), jnp.float32)
    for g in range(_RPW // 16):
        ridx = riota + g * 16

        def _dbody(d, carry):
            du, dt = carry
            dvec = jnp.broadcast_to(d, (16,))
            fidx = ridx * _D + dvec
            gu = plsc.load_gather(accu, [fidx])
            gw = plsc.load_gather(accw, [fidx])
            us = plsc.load_gather(usv, [dvec])
            return (du + gu * gw, dt + us * gw)

        du, dt = lax.fori_loop(0, _D, _dbody, (zero16, zero16))
        sl = pl.ds(g * 16, 16)
        nf = nv[sl].astype(jnp.float32)
        lf = lv[sl].astype(jnp.float32)
        sv[sl] = du / (nf * lf)
        spv[sl] = (dt - du) / ((float(_N_USERS) - nf) * lf)

    pltpu.sync_copy(sv, s_hbm.at[pl.ds(base, _RPW)])
    pltpu.sync_copy(spv, sp_hbm.at[pl.ds(base, _RPW)])


_SCRATCH_W = [
    pltpu.VMEM((_KW, _RPW), jnp.int32),
    pltpu.VMEM((2, _JC, _RPW, _D), jnp.float32),
    pltpu.VMEM((_RPW * _D,), jnp.float32),
    pltpu.SemaphoreType.DMA,
]

_SCRATCH_U = [
    pltpu.VMEM((_KU, _RPW), jnp.int32),
    pltpu.VMEM((2, _JC, _RPW, _D), jnp.float32),
    pltpu.VMEM((_RPW * _D,), jnp.float32),
    pltpu.VMEM((_RPW * _D,), jnp.float32),
    pltpu.VMEM((_RPW,), jnp.int32),
    pltpu.VMEM((_RPW,), jnp.int32),
    pltpu.VMEM((_D,), jnp.float32),
    pltpu.VMEM((_RPW,), jnp.float32),
    pltpu.VMEM((_RPW,), jnp.float32),
    pltpu.SemaphoreType.DMA,
]


def kernel(ui, wi, l, n, U, W):
    w_lin = _convw(W.T)
    scw = pl.kernel(
        _scw_body,
        out_type=jax.ShapeDtypeStruct((_B * _D,), jnp.float32),
        scratch_types=_SCRATCH_W,
        **_SC_MESH,
    )
    accw = scw(wi.T, w_lin.reshape(_NPAD, _D))

    u_lin, usum = _convu(U.T)
    scu = pl.kernel(
        _scu_body,
        out_type=(jax.ShapeDtypeStruct((_B,), jnp.float32),
                  jax.ShapeDtypeStruct((_B,), jnp.float32)),
        scratch_types=_SCRATCH_U,
        **_SC_MESH,
    )
    s, sp = scu(ui.T, l, n, u_lin.reshape(_NPAD, _D), usum.reshape(_D), accw)
    return (s, sp)
